# Initial kernel scaffold; baseline (speedup 1.0000x reference)
import jax
import jax.numpy as jnp
from jax.experimental import pallas as pl


def kernel(x, edge_index, edge_attr, u, batch, WK, bK, WQ, bQ):
    raise NotImplementedError("write your pallas kernel here")


# speedup vs baseline: 4.2552x; 4.2552x over previous
import functools

import jax
import jax.numpy as jnp
from jax import lax
from jax.experimental import pallas as pl
from jax.experimental.pallas import tpu as pltpu
from jax.experimental.pallas import tpu_sc as plsc

N_NODES = 10000
N_EDGES = 320000
F = 128
G = 16

NC = 2
NS = 16
NW = NC * NS
EDGES_PER_W = N_EDGES // NW
CHUNK = 80
NCHUNKS = EDGES_PER_W // CHUNK
ROWS_PER_TILE = 624
ROWS_TAIL = N_NODES - NS * ROWS_PER_TILE
NVEC = F // 16


def _bcast_lane(v, i):
    idx = lax.full((16, 1), i, jnp.int32)
    return lax.gather(
        v, idx,
        lax.GatherDimensionNumbers(
            offset_dims=(), collapsed_slice_dims=(0,), start_index_map=(0,)),
        (1,), mode=lax.GatherScatterMode.PROMISE_IN_BOUNDS)


def _sc_agg_body(x_hbm, src_hbm, dst_hbm, attr_hbm, out_hbm,
                 acc, srcv, dstv, attrv, rows, sem):
    c = lax.axis_index("c")
    s = lax.axis_index("s")
    wid = s * NC + c
    ebase = wid * EDGES_PER_W

    def _zero_row(r, _):
        for j in range(NVEC):
            rows[r, pl.ds(j * 16, 16)] = jnp.zeros((16,), jnp.float32)
        return _
    lax.fori_loop(0, CHUNK, _zero_row, 0)
    rbase = s * ROWS_PER_TILE
    for t in range(ROWS_PER_TILE // CHUNK):
        pltpu.sync_copy(rows, acc.at[pl.ds(rbase + t * CHUNK, CHUNK)])
    _rem = ROWS_PER_TILE % CHUNK
    if _rem:
        pltpu.sync_copy(
            rows.at[pl.ds(0, _rem)],
            acc.at[pl.ds(rbase + (ROWS_PER_TILE // CHUNK) * CHUNK, _rem)])

    @pl.when(s == 0)
    def _zero_tail():
        pltpu.sync_copy(rows.at[pl.ds(0, ROWS_TAIL)],
                        acc.at[pl.ds(NS * ROWS_PER_TILE, ROWS_TAIL)])
    plsc.subcore_barrier()

    def _chunk(k, _):
        base = ebase + k * CHUNK
        pltpu.sync_copy(src_hbm.at[pl.ds(base, CHUNK)], srcv)
        pltpu.sync_copy(dst_hbm.at[pl.ds(base, CHUNK)], dstv)
        pltpu.sync_copy(attr_hbm.at[pl.ds(base, CHUNK)], attrv)
        pltpu.async_copy(x_hbm.at[srcv], rows, sem).wait()

        def _group(g, _2):
            av = attrv[pl.ds(g * 16, 16)]
            for i in range(16):
                sc = _bcast_lane(av, i)
                e = g * 16 + i
                for j in range(NVEC):
                    sl = pl.ds(j * 16, 16)
                    rows[e, sl] = rows[e, sl] * sc
            return _2
        lax.fori_loop(0, CHUNK // 16, _group, 0)

        pltpu.sync_copy(rows, acc.at[dstv], add=True)
        return _
    lax.fori_loop(0, NCHUNKS, _chunk, 0)

    plsc.subcore_barrier()
    pltpu.sync_copy(acc.at[pl.ds(rbase, ROWS_PER_TILE)],
                    out_hbm.at[c, pl.ds(rbase, ROWS_PER_TILE)])

    @pl.when(s == 0)
    def _out_tail():
        pltpu.sync_copy(acc.at[pl.ds(NS * ROWS_PER_TILE, ROWS_TAIL)],
                        out_hbm.at[c, pl.ds(NS * ROWS_PER_TILE, ROWS_TAIL)])


@jax.jit
def _sc_agg(x, src, dst, attr):
    mesh = plsc.VectorSubcoreMesh(core_axis_name="c", subcore_axis_name="s")
    return pl.kernel(
        _sc_agg_body,
        out_type=jax.ShapeDtypeStruct((NC, N_NODES, F), jnp.float32),
        mesh=mesh,
        scratch_types=[
            pltpu.VMEM_SHARED((N_NODES, F), jnp.float32),
            pltpu.VMEM((CHUNK,), jnp.int32),
            pltpu.VMEM((CHUNK,), jnp.int32),
            pltpu.VMEM((CHUNK,), jnp.float32),
            pltpu.VMEM((CHUNK, F), jnp.float32),
            pltpu.SemaphoreType.DMA,
        ],
    )(x, src, dst, attr)


ROWBLK = 400
NBLK = N_NODES // ROWBLK


def _tc_body(agg_ref, b_ref, u_ref, wk1, wk2, wq1, wq2, bk, bq, k_ref, q_ref):
    xa = agg_ref[0] + agg_ref[1]
    oh = (b_ref[...] == lax.broadcasted_iota(jnp.int32, (ROWBLK, G), 1)
          ).astype(jnp.float32)
    ub = jnp.dot(oh, u_ref[...], precision=lax.Precision.HIGHEST)
    hp = lax.Precision.HIGHEST
    k_ref[...] = (jnp.dot(xa, wk1[...], precision=hp)
                  + jnp.dot(ub, wk2[...], precision=hp) + bk[...])
    q_ref[...] = (jnp.dot(xa, wq1[...], precision=hp)
                  + jnp.dot(ub, wq2[...], precision=hp) + bq[...])


@jax.jit
def _tc_linear(agg, batch2d, u, wk1, wk2, wq1, wq2, bk, bq):
    full = lambda *shape: pl.BlockSpec(shape, lambda i: tuple(0 for _ in shape))
    return pl.pallas_call(
        _tc_body,
        grid=(NBLK,),
        in_specs=[
            pl.BlockSpec((NC, ROWBLK, F), lambda i: (0, i, 0)),
            pl.BlockSpec((ROWBLK, 1), lambda i: (i, 0)),
            full(G, F),
            full(F, F), full(F, F), full(F, F), full(F, F),
            full(1, F), full(1, F),
        ],
        out_specs=[
            pl.BlockSpec((ROWBLK, F), lambda i: (i, 0)),
            pl.BlockSpec((ROWBLK, F), lambda i: (i, 0)),
        ],
        out_shape=[
            jax.ShapeDtypeStruct((N_NODES, F), jnp.float32),
            jax.ShapeDtypeStruct((N_NODES, F), jnp.float32),
        ],
    )(agg, batch2d, u, wk1, wk2, wq1, wq2, bk, bq)


def kernel(x, edge_index, edge_attr, u, batch, WK, bK, WQ, bQ):
    src = edge_index[0].astype(jnp.int32)
    dst = edge_index[1].astype(jnp.int32)
    attr = edge_attr.reshape(N_EDGES)
    agg = _sc_agg(x, src, dst, attr)
    batch2d = batch.astype(jnp.int32).reshape(N_NODES, 1)
    K, Q = _tc_linear(
        agg, batch2d, u,
        WK[:, :F].T, WK[:, F:].T, WQ[:, :F].T, WQ[:, F:].T,
        bK.reshape(1, F), bQ.reshape(1, F))
    return K, Q


# speedup vs baseline: 10.4680x; 2.4601x over previous
import functools

import jax
import jax.numpy as jnp
from jax import lax
from jax.experimental import pallas as pl
from jax.experimental.pallas import tpu as pltpu
from jax.experimental.pallas import tpu_sc as plsc

N_NODES = 10000
N_EDGES = 320000
F = 128
G = 16

NC = 2
NS = 16
NW = NC * NS
EDGES_PER_W = N_EDGES // NW
CHUNK = 80
NCHUNKS = EDGES_PER_W // CHUNK
NRING = 4
NMAIN = NCHUNKS - 1
ROWS_PER_TILE = 624
ROWS_TAIL = N_NODES - NS * ROWS_PER_TILE
NVEC = F // 16


def _bcast_lane(v, i):
    idx = lax.full((16, 1), i, jnp.int32)
    return lax.gather(
        v, idx,
        lax.GatherDimensionNumbers(
            offset_dims=(), collapsed_slice_dims=(0,), start_index_map=(0,)),
        (1,), mode=lax.GatherScatterMode.PROMISE_IN_BOUNDS)


def _sc_agg_body(x_hbm, src_hbm, dst_hbm, attr_hbm, out_hbm,
                 acc, srcv, dstv, attrv, rows, sem_g, sem_s, sem_ia, sem_id):
    c = lax.axis_index("c")
    s = lax.axis_index("s")
    wid = s * NC + c
    ebase = wid * EDGES_PER_W

    def _zero_row(r, carry):
        for j in range(NVEC):
            rows[0][r, pl.ds(j * 16, 16)] = jnp.zeros((16,), jnp.float32)
        return carry
    lax.fori_loop(0, CHUNK, _zero_row, 0)
    rbase = s * ROWS_PER_TILE
    for t in range(ROWS_PER_TILE // CHUNK):
        pltpu.sync_copy(rows[0], acc.at[pl.ds(rbase + t * CHUNK, CHUNK)])
    _rem = ROWS_PER_TILE % CHUNK
    if _rem:
        pltpu.sync_copy(
            rows[0].at[pl.ds(0, _rem)],
            acc.at[pl.ds(rbase + (ROWS_PER_TILE // CHUNK) * CHUNK, _rem)])

    @pl.when(s == 0)
    def _zero_tail():
        pltpu.sync_copy(rows[0].at[pl.ds(0, ROWS_TAIL)],
                        acc.at[pl.ds(NS * ROWS_PER_TILE, ROWS_TAIL)])
    plsc.subcore_barrier()

    def issue_ia(k, b):
        base = ebase + k * CHUNK
        pltpu.async_copy(src_hbm.at[pl.ds(base, CHUNK)], srcv[b], sem_ia[b])
        pltpu.async_copy(attr_hbm.at[pl.ds(base, CHUNK)], attrv[b], sem_ia[b])

    def wait_ia(b):
        pltpu.make_async_copy(src_hbm.at[pl.ds(0, CHUNK)], srcv[b],
                              sem_ia[b]).wait()
        pltpu.make_async_copy(attr_hbm.at[pl.ds(0, CHUNK)], attrv[b],
                              sem_ia[b]).wait()

    def issue_id(k, b):
        base = ebase + k * CHUNK
        pltpu.async_copy(dst_hbm.at[pl.ds(base, CHUNK)], dstv[b], sem_id[b])

    def wait_id(b):
        pltpu.make_async_copy(dst_hbm.at[pl.ds(0, CHUNK)], dstv[b],
                              sem_id[b]).wait()

    def issue_gather(b):
        pltpu.async_copy(x_hbm.at[srcv[b]], rows[b], sem_g[b])

    def wait_gather(b):
        pltpu.make_async_copy(x_hbm.at[srcv[b]], rows[b], sem_g[b]).wait()

    def issue_scatter(b):
        pltpu.async_copy(rows[b], acc.at[dstv[b]], sem_s[b], add=True)

    def wait_scatter(b):
        pltpu.make_async_copy(rows[b], acc.at[dstv[b]], sem_s[b]).wait()

    def multiply(b):
        def _group(g, carry):
            av = attrv[b][pl.ds(g * 16, 16)]
            for i in range(16):
                sc = _bcast_lane(av, i)
                e = g * 16 + i
                for j in range(NVEC):
                    sl = pl.ds(j * 16, 16)
                    rows[b][e, sl] = rows[b][e, sl] * sc
            return carry
        lax.fori_loop(0, CHUNK // 16, _group, 0)

    issue_ia(0, 0)
    issue_ia(1, 1)
    issue_ia(2, 2)
    issue_id(0, 0)
    wait_ia(0)
    issue_gather(0)
    wait_ia(1)
    issue_gather(1)

    def _super(it, carry):
        for r in range(NRING):
            k = it * NRING + r
            p = r
            @pl.when(k >= 2)
            def _w():
                wait_scatter((r + 2) % NRING)
            @pl.when(k + 3 < NCHUNKS)
            def _b1():
                issue_ia(k + 3, (r + 3) % NRING)
            @pl.when(k + 1 < NCHUNKS)
            def _b2():
                issue_id(k + 1, (r + 1) % NRING)
            wait_gather(p)
            multiply(p)
            wait_id(p)
            issue_scatter(p)
            @pl.when(k + 2 < NCHUNKS)
            def _f():
                wait_ia((r + 2) % NRING)
                issue_gather((r + 2) % NRING)
        return carry
    lax.fori_loop(0, NMAIN // NRING, _super, 0)

    wait_scatter(2)
    wait_gather(0)
    multiply(0)
    wait_id(0)
    issue_scatter(0)
    wait_scatter(3)
    wait_scatter(0)

    plsc.subcore_barrier()
    pltpu.sync_copy(acc.at[pl.ds(rbase, ROWS_PER_TILE)],
                    out_hbm.at[c, pl.ds(rbase, ROWS_PER_TILE)])

    @pl.when(s == 0)
    def _out_tail():
        pltpu.sync_copy(acc.at[pl.ds(NS * ROWS_PER_TILE, ROWS_TAIL)],
                        out_hbm.at[c, pl.ds(NS * ROWS_PER_TILE, ROWS_TAIL)])


@jax.jit
def _sc_agg(x, src, dst, attr):
    mesh = plsc.VectorSubcoreMesh(core_axis_name="c", subcore_axis_name="s")
    return pl.kernel(
        _sc_agg_body,
        out_type=jax.ShapeDtypeStruct((NC, N_NODES, F), jnp.float32),
        mesh=mesh,
        scratch_types=[
            pltpu.VMEM_SHARED((N_NODES, F), jnp.float32),
            [pltpu.VMEM((CHUNK,), jnp.int32) for _ in range(NRING)],
            [pltpu.VMEM((CHUNK,), jnp.int32) for _ in range(NRING)],
            [pltpu.VMEM((CHUNK,), jnp.float32) for _ in range(NRING)],
            [pltpu.VMEM((CHUNK, F), jnp.float32) for _ in range(NRING)],
            [pltpu.SemaphoreType.DMA for _ in range(NRING)],
            [pltpu.SemaphoreType.DMA for _ in range(NRING)],
            [pltpu.SemaphoreType.DMA for _ in range(NRING)],
            [pltpu.SemaphoreType.DMA for _ in range(NRING)],
        ],
    )(x, src, dst, attr)


ROWBLK = 400
NBLK = N_NODES // ROWBLK


def _tc_body(agg_ref, b_ref, u_ref, wk1, wk2, wq1, wq2, bk, bq, k_ref, q_ref):
    xa = agg_ref[0] + agg_ref[1]
    oh = (b_ref[...] == lax.broadcasted_iota(jnp.int32, (ROWBLK, G), 1)
          ).astype(jnp.float32)
    hp = lax.Precision.HIGHEST
    ub = jnp.dot(oh, u_ref[...], precision=hp)
    k_ref[...] = (jnp.dot(xa, wk1[...], precision=hp)
                  + jnp.dot(ub, wk2[...], precision=hp) + bk[...])
    q_ref[...] = (jnp.dot(xa, wq1[...], precision=hp)
                  + jnp.dot(ub, wq2[...], precision=hp) + bq[...])


@jax.jit
def _tc_linear(agg, batch2d, u, wk1, wk2, wq1, wq2, bk, bq):
    full = lambda *shape: pl.BlockSpec(shape, lambda i: tuple(0 for _ in shape))
    return pl.pallas_call(
        _tc_body,
        grid=(NBLK,),
        in_specs=[
            pl.BlockSpec((NC, ROWBLK, F), lambda i: (0, i, 0)),
            pl.BlockSpec((ROWBLK, 1), lambda i: (i, 0)),
            full(G, F),
            full(F, F), full(F, F), full(F, F), full(F, F),
            full(1, F), full(1, F),
        ],
        out_specs=[
            pl.BlockSpec((ROWBLK, F), lambda i: (i, 0)),
            pl.BlockSpec((ROWBLK, F), lambda i: (i, 0)),
        ],
        out_shape=[
            jax.ShapeDtypeStruct((N_NODES, F), jnp.float32),
            jax.ShapeDtypeStruct((N_NODES, F), jnp.float32),
        ],
    )(agg, batch2d, u, wk1, wk2, wq1, wq2, bk, bq)


def kernel(x, edge_index, edge_attr, u, batch, WK, bK, WQ, bQ):
    src = edge_index[0].astype(jnp.int32)
    dst = edge_index[1].astype(jnp.int32)
    attr = edge_attr.reshape(N_EDGES)
    agg = _sc_agg(x, src, dst, attr)
    batch2d = batch.astype(jnp.int32).reshape(N_NODES, 1)
    K, Q = _tc_linear(
        agg, batch2d, u,
        WK[:, :F].T, WK[:, F:].T, WQ[:, :F].T, WQ[:, F:].T,
        bK.reshape(1, F), bQ.reshape(1, F))
    return K, Q
